# baseline (speedup 1.0000x reference)
import functools

import jax
import jax.numpy as jnp
from jax import lax
from jax.experimental import pallas as pl
from jax.experimental.pallas import tpu as pltpu

B = 4096
D_MODEL = 2048
D_SPACE = 64
N_R = 64
NEURONS = 64
BT = 512


def _softmax_last(v):
    m = jnp.max(v, axis=-1, keepdims=True)
    e = jnp.exp(v - m)
    return e / jnp.sum(e, axis=-1, keepdims=True)


def _prep_body(ce_ref, ne_ref, en_ref, nn_ref, red_ref):
    ce = ce_ref[...]
    en_ref[...] = ce / (jnp.sqrt(jnp.sum(ce * ce, axis=-1, keepdims=True)) + 1e-12)
    ne = ne_ref[...]
    nn_ref[...] = ne / (jnp.sqrt(jnp.sum(ne * ne, axis=-1, keepdims=True)) + 1e-12)
    jj = lax.broadcasted_iota(jnp.int32, (NEURONS * N_R, NEURONS), 0)
    nn_idx = lax.broadcasted_iota(jnp.int32, (NEURONS * N_R, NEURONS), 1)
    red_ref[...] = jnp.where((jj // N_R) == nn_idx, 1.0, 0.0).astype(jnp.float32)


def _body(x_ref, wcat_ref, en_ref, nn_ref, red_ref, out_ref):
    x = x_ref[...]
    hcat = lax.dot_general(x, wcat_ref[...], (((1,), (1,)), ((), ())),
                           preferred_element_type=jnp.float32)
    h = hcat[:, :D_SPACE]
    hn = hcat[:, D_SPACE:]

    logits = lax.dot_general(h, en_ref[...], (((1,), (1,)), ((), ())),
                             preferred_element_type=jnp.float32)

    iota = lax.broadcasted_iota(jnp.int32, logits.shape, 1)
    v1 = jnp.max(logits, axis=-1, keepdims=True)
    i1 = jnp.min(jnp.where(logits == v1, iota, N_R), axis=-1, keepdims=True)
    one1 = iota == i1
    masked = jnp.where(one1, -jnp.inf, logits)
    v2 = jnp.max(masked, axis=-1, keepdims=True)
    i2 = jnp.min(jnp.where(masked == v2, iota, N_R), axis=-1, keepdims=True)

    e = jnp.exp(v2 - v1)
    g1 = 1.0 / (1.0 + e)
    g2 = e / (1.0 + e)

    full = lax.dot_general(hn, nn_ref[...], (((1,), (1,)), ((), ())),
                           preferred_element_type=jnp.float32)

    col = lax.broadcasted_iota(jnp.int32, (BT, NEURONS * N_R), 1)
    c_of_col = col & (N_R - 1)
    red = red_ref[...]
    nl1 = lax.dot_general(jnp.where(c_of_col == i1, full, 0.0), red,
                          (((1,), (0,)), ((), ())),
                          preferred_element_type=jnp.float32)
    nl2 = lax.dot_general(jnp.where(c_of_col == i2, full, 0.0), red,
                          (((1,), (0,)), ((), ())),
                          preferred_element_type=jnp.float32)

    nl = jnp.concatenate([nl1, nl2], axis=0)
    w = _softmax_last(nl) * jnp.concatenate([g1, g2], axis=0)
    out_ref[:, 0, :] = w[:BT]
    out_ref[:, 1, :] = w[BT:]


@functools.partial(jax.jit, static_argnames=())
def _run(x, wcat, ce_r, ne_t):
    en, nn, red = pl.pallas_call(
        _prep_body,
        out_shape=[
            jax.ShapeDtypeStruct((N_R, D_SPACE), jnp.float32),
            jax.ShapeDtypeStruct((NEURONS * N_R, D_SPACE), jnp.float32),
            jax.ShapeDtypeStruct((NEURONS * N_R, NEURONS), jnp.float32),
        ],
    )(ce_r, ne_t)
    grid = (B // BT,)
    return pl.pallas_call(
        _body,
        grid=grid,
        in_specs=[
            pl.BlockSpec((BT, D_MODEL), lambda i: (i, 0)),
            pl.BlockSpec((2 * D_SPACE, D_MODEL), lambda i: (0, 0)),
            pl.BlockSpec((N_R, D_SPACE), lambda i: (0, 0)),
            pl.BlockSpec((NEURONS * N_R, D_SPACE), lambda i: (0, 0)),
            pl.BlockSpec((NEURONS * N_R, NEURONS), lambda i: (0, 0)),
        ],
        out_specs=pl.BlockSpec((BT, 2, NEURONS), lambda i: (i, 0, 0)),
        out_shape=jax.ShapeDtypeStruct((B, 2, NEURONS), jnp.float32),
        compiler_params=pltpu.CompilerParams(
            dimension_semantics=("arbitrary",),
        ),
    )(x, wcat, en, nn, red)


def kernel(x, neuron_emb, W_proj, W_neuron, circuit_emb, top_k):
    del top_k
    wcat = jnp.concatenate([W_proj, W_neuron], axis=0)
    ce_r = circuit_emb[:N_R]
    ne_t = jnp.transpose(neuron_emb, (1, 0, 2)).reshape(NEURONS * N_R, D_SPACE)
    return _run(x, wcat, ce_r, ne_t)


# speedup vs baseline: 1.0578x; 1.0578x over previous
import functools

import jax
import jax.numpy as jnp
from jax import lax
from jax.experimental import pallas as pl
from jax.experimental.pallas import tpu as pltpu

B = 4096
D_MODEL = 2048
D_SPACE = 64
N_R = 64
NEURONS = 64
BT = 512


def _softmax_last(v):
    m = jnp.max(v, axis=-1, keepdims=True)
    e = jnp.exp(v - m)
    return e / jnp.sum(e, axis=-1, keepdims=True)


def _body(x_ref, wcat_ref, ce_ref, ne_ref, out_ref, en_ref, nn_ref, red_ref):
    i = pl.program_id(0)

    @pl.when(i == 0)
    def _init():
        ce = ce_ref[...]
        en_ref[...] = ce / (jnp.sqrt(jnp.sum(ce * ce, axis=-1, keepdims=True)) + 1e-12)
        ne = ne_ref[...]
        nn_ref[...] = ne / (jnp.sqrt(jnp.sum(ne * ne, axis=-1, keepdims=True)) + 1e-12)
        jj = lax.broadcasted_iota(jnp.int32, (NEURONS * N_R, NEURONS), 0)
        nn_idx = lax.broadcasted_iota(jnp.int32, (NEURONS * N_R, NEURONS), 1)
        red_ref[...] = jnp.where((jj // N_R) == nn_idx, 1.0, 0.0).astype(jnp.float32)

    x = x_ref[...]
    hcat = lax.dot_general(x, wcat_ref[...], (((1,), (1,)), ((), ())),
                           preferred_element_type=jnp.float32)
    h = hcat[:, :D_SPACE]
    hn = hcat[:, D_SPACE:]

    logits = lax.dot_general(h, en_ref[...], (((1,), (1,)), ((), ())),
                             preferred_element_type=jnp.float32)

    iota = lax.broadcasted_iota(jnp.int32, logits.shape, 1)
    v1 = jnp.max(logits, axis=-1, keepdims=True)
    i1 = jnp.min(jnp.where(logits == v1, iota, N_R), axis=-1, keepdims=True)
    one1 = iota == i1
    masked = jnp.where(one1, -jnp.inf, logits)
    v2 = jnp.max(masked, axis=-1, keepdims=True)
    i2 = jnp.min(jnp.where(masked == v2, iota, N_R), axis=-1, keepdims=True)

    e = jnp.exp(v2 - v1)
    g1 = 1.0 / (1.0 + e)
    g2 = e / (1.0 + e)

    full = lax.dot_general(hn, nn_ref[...], (((1,), (1,)), ((), ())),
                           preferred_element_type=jnp.float32)

    col = lax.broadcasted_iota(jnp.int32, (BT, NEURONS * N_R), 1)
    c_of_col = col & (N_R - 1)
    red = red_ref[...]
    nl1 = lax.dot_general(jnp.where(c_of_col == i1, full, 0.0), red,
                          (((1,), (0,)), ((), ())),
                          preferred_element_type=jnp.float32)
    nl2 = lax.dot_general(jnp.where(c_of_col == i2, full, 0.0), red,
                          (((1,), (0,)), ((), ())),
                          preferred_element_type=jnp.float32)

    nl = jnp.concatenate([nl1, nl2], axis=0)
    w = _softmax_last(nl) * jnp.concatenate([g1, g2], axis=0)
    out_ref[:, 0, :] = w[:BT]
    out_ref[:, 1, :] = w[BT:]


@functools.partial(jax.jit, static_argnames=())
def _run(x, wcat, ce_r, ne_t):
    grid = (B // BT,)
    return pl.pallas_call(
        _body,
        grid=grid,
        in_specs=[
            pl.BlockSpec((BT, D_MODEL), lambda i: (i, 0)),
            pl.BlockSpec((2 * D_SPACE, D_MODEL), lambda i: (0, 0)),
            pl.BlockSpec((N_R, D_SPACE), lambda i: (0, 0)),
            pl.BlockSpec((NEURONS * N_R, D_SPACE), lambda i: (0, 0)),
        ],
        out_specs=pl.BlockSpec((BT, 2, NEURONS), lambda i: (i, 0, 0)),
        out_shape=jax.ShapeDtypeStruct((B, 2, NEURONS), jnp.float32),
        scratch_shapes=[
            pltpu.VMEM((N_R, D_SPACE), jnp.float32),
            pltpu.VMEM((NEURONS * N_R, D_SPACE), jnp.float32),
            pltpu.VMEM((NEURONS * N_R, NEURONS), jnp.float32),
        ],
        compiler_params=pltpu.CompilerParams(
            dimension_semantics=("arbitrary",),
        ),
    )(x, wcat, ce_r, ne_t)


def kernel(x, neuron_emb, W_proj, W_neuron, circuit_emb, top_k):
    del top_k
    wcat = jnp.concatenate([W_proj, W_neuron], axis=0)
    ce_r = circuit_emb[:N_R]
    ne_t = jnp.transpose(neuron_emb, (1, 0, 2)).reshape(NEURONS * N_R, D_SPACE)
    return _run(x, wcat, ce_r, ne_t)


# speedup vs baseline: 1.1498x; 1.0870x over previous
import functools

import jax
import jax.numpy as jnp
from jax import lax
from jax.experimental import pallas as pl
from jax.experimental.pallas import tpu as pltpu

B = 4096
D_MODEL = 2048
D_SPACE = 64
N_R = 64
NEURONS = 64
BT = 512


def _softmax_last(v):
    m = jnp.max(v, axis=-1, keepdims=True)
    e = jnp.exp(v - m)
    return e / jnp.sum(e, axis=-1, keepdims=True)


def _body(x_ref, wp_ref, wn_ref, ce_ref, ne_ref, out_ref, wcat_ref, en_ref,
          nn_ref, red_ref):
    i = pl.program_id(0)

    @pl.when(i == 0)
    def _init():
        wcat_ref[:D_SPACE, :] = wp_ref[...]
        wcat_ref[D_SPACE:, :] = wn_ref[...]
        ce = ce_ref[...]
        en_ref[...] = ce / (jnp.sqrt(jnp.sum(ce * ce, axis=-1, keepdims=True)) + 1e-12)
        ne = ne_ref[...]
        nn_ref[...] = ne / (jnp.sqrt(jnp.sum(ne * ne, axis=-1, keepdims=True)) + 1e-12)
        jj = lax.broadcasted_iota(jnp.int32, (NEURONS * N_R, NEURONS), 0)
        nn_idx = lax.broadcasted_iota(jnp.int32, (NEURONS * N_R, NEURONS), 1)
        red_ref[...] = jnp.where((jj & (NEURONS - 1)) == nn_idx, 1.0, 0.0).astype(jnp.float32)

    x = x_ref[...]
    hcat = lax.dot_general(x, wcat_ref[...], (((1,), (1,)), ((), ())),
                           preferred_element_type=jnp.float32)
    h = hcat[:, :D_SPACE]
    hn = hcat[:, D_SPACE:]

    logits = lax.dot_general(h, en_ref[...], (((1,), (1,)), ((), ())),
                             preferred_element_type=jnp.float32)

    iota = lax.broadcasted_iota(jnp.int32, logits.shape, 1)
    v1 = jnp.max(logits, axis=-1, keepdims=True)
    i1 = jnp.min(jnp.where(logits == v1, iota, N_R), axis=-1, keepdims=True)
    one1 = iota == i1
    masked = jnp.where(one1, -jnp.inf, logits)
    v2 = jnp.max(masked, axis=-1, keepdims=True)
    i2 = jnp.min(jnp.where(masked == v2, iota, N_R), axis=-1, keepdims=True)

    e = jnp.exp(v2 - v1)
    g1 = 1.0 / (1.0 + e)
    g2 = e / (1.0 + e)

    full = lax.dot_general(hn, nn_ref[...], (((1,), (1,)), ((), ())),
                           preferred_element_type=jnp.float32)

    col = lax.broadcasted_iota(jnp.int32, (BT, NEURONS * N_R), 1)
    c_of_col = col // NEURONS
    red = red_ref[...]
    nl1 = lax.dot_general(jnp.where(c_of_col == i1, full, 0.0), red,
                          (((1,), (0,)), ((), ())),
                          preferred_element_type=jnp.float32)
    nl2 = lax.dot_general(jnp.where(c_of_col == i2, full, 0.0), red,
                          (((1,), (0,)), ((), ())),
                          preferred_element_type=jnp.float32)

    nl = jnp.concatenate([nl1, nl2], axis=0)
    w = _softmax_last(nl) * jnp.concatenate([g1, g2], axis=0)
    out_ref[:, 0, :] = w[:BT]
    out_ref[:, 1, :] = w[BT:]


@functools.partial(jax.jit, static_argnames=())
def _run(x, wp, wn, ce, ne_flat):
    grid = (B // BT,)
    return pl.pallas_call(
        _body,
        grid=grid,
        in_specs=[
            pl.BlockSpec((BT, D_MODEL), lambda i: (i, 0)),
            pl.BlockSpec((D_SPACE, D_MODEL), lambda i: (0, 0)),
            pl.BlockSpec((D_SPACE, D_MODEL), lambda i: (0, 0)),
            pl.BlockSpec((N_R, D_SPACE), lambda i: (0, 0)),
            pl.BlockSpec((NEURONS * N_R, D_SPACE), lambda i: (0, 0)),
        ],
        out_specs=pl.BlockSpec((BT, 2, NEURONS), lambda i: (i, 0, 0)),
        out_shape=jax.ShapeDtypeStruct((B, 2, NEURONS), jnp.float32),
        scratch_shapes=[
            pltpu.VMEM((2 * D_SPACE, D_MODEL), jnp.float32),
            pltpu.VMEM((N_R, D_SPACE), jnp.float32),
            pltpu.VMEM((NEURONS * N_R, D_SPACE), jnp.float32),
            pltpu.VMEM((NEURONS * N_R, NEURONS), jnp.float32),
        ],
        compiler_params=pltpu.CompilerParams(
            dimension_semantics=("arbitrary",),
        ),
    )(x, wp, wn, ce, ne_flat)


def kernel(x, neuron_emb, W_proj, W_neuron, circuit_emb, top_k):
    del top_k
    ne_flat = neuron_emb.reshape(N_R * NEURONS, D_SPACE)
    return _run(x, W_proj, W_neuron, circuit_emb, ne_flat)
